# baseline (speedup 1.0000x reference)
import jax
import jax.numpy as jnp
from jax.experimental import pallas as pl
from jax.experimental.pallas import tpu as pltpu

ROWS, COLS = 16384, 312
OUT_COLS = 112


def _body(x_ref, o_ref, sem):
    copies = [
        pltpu.make_async_copy(x_ref.at[2 * j], o_ref.at[j], sem)
        for j in range(OUT_COLS)
    ]
    for c in copies:
        c.start()
    for c in copies:
        c.wait()


@jax.jit
def kernel(x):
    xt = x.T
    out_t = pl.pallas_call(
        _body,
        in_specs=[pl.BlockSpec(memory_space=pl.ANY)],
        out_specs=pl.BlockSpec(memory_space=pl.ANY),
        out_shape=jax.ShapeDtypeStruct((OUT_COLS, ROWS), jnp.float32),
        scratch_shapes=[pltpu.SemaphoreType.DMA],
    )(xt)
    return out_t.T


# speedup vs baseline: 37.0511x; 37.0511x over previous
import jax
import jax.numpy as jnp
from jax.experimental import pallas as pl
from jax.experimental.pallas import tpu as pltpu

ROWS, COLS = 16384, 312
OUT_COLS = 112
NSEM = 8


def _body(x_ref, o_ref, buf, sems, osem):
    copies = [
        pltpu.make_async_copy(x_ref.at[2 * j], buf.at[j], sems.at[j % NSEM])
        for j in range(OUT_COLS)
    ]
    for c in copies:
        c.start()
    for c in copies:
        c.wait()
    out = pltpu.make_async_copy(buf, o_ref, osem)
    out.start()
    out.wait()


@jax.jit
def kernel(x):
    xt = x.T
    out_t = pl.pallas_call(
        _body,
        in_specs=[pl.BlockSpec(memory_space=pl.ANY)],
        out_specs=pl.BlockSpec(memory_space=pl.ANY),
        out_shape=jax.ShapeDtypeStruct((OUT_COLS, ROWS), jnp.float32),
        scratch_shapes=[
            pltpu.VMEM((OUT_COLS, ROWS), jnp.float32),
            pltpu.SemaphoreType.DMA((NSEM,)),
            pltpu.SemaphoreType.DMA,
        ],
    )(xt)
    return out_t.T


# speedup vs baseline: 42.2799x; 1.1411x over previous
import jax
import jax.numpy as jnp
from jax.experimental import pallas as pl
from jax.experimental.pallas import tpu as pltpu

ROWS, COLS = 16384, 312
OUT_COLS = 112
CHUNK = 16
NCHUNK = OUT_COLS // CHUNK


def _body(x_ref, o_ref, buf, isems, osems):
    copies = [
        pltpu.make_async_copy(x_ref.at[2 * j], buf.at[j], isems.at[j // CHUNK])
        for j in range(OUT_COLS)
    ]
    for c in copies:
        c.start()
    stores = []
    for ck in range(NCHUNK):
        for j in range(ck * CHUNK, (ck + 1) * CHUNK):
            copies[j].wait()
        st = pltpu.make_async_copy(
            buf.at[pl.ds(ck * CHUNK, CHUNK)],
            o_ref.at[pl.ds(ck * CHUNK, CHUNK)],
            osems.at[ck],
        )
        st.start()
        stores.append(st)
    for st in stores:
        st.wait()


@jax.jit
def kernel(x):
    xt = x.T
    out_t = pl.pallas_call(
        _body,
        in_specs=[pl.BlockSpec(memory_space=pl.ANY)],
        out_specs=pl.BlockSpec(memory_space=pl.ANY),
        out_shape=jax.ShapeDtypeStruct((OUT_COLS, ROWS), jnp.float32),
        scratch_shapes=[
            pltpu.VMEM((OUT_COLS, ROWS), jnp.float32),
            pltpu.SemaphoreType.DMA((NCHUNK,)),
            pltpu.SemaphoreType.DMA((NCHUNK,)),
        ],
    )(xt)
    return out_t.T
